# baseline (speedup 1.0000x reference)
import functools

import jax
import jax.numpy as jnp
from jax import lax
from jax.experimental import pallas as pl
from jax.experimental.pallas import tpu as pltpu
from jax.experimental.pallas import tpu_sc as plsc

NUM_ACTIONS = 1000
EMBED_DIM = 64
BATCH = 16384
HIST = 50

NC = 2
NS = 16
NW = NC * NS
LANES = 16

N_FLAT = BATCH * HIST
PER_W = N_FLAT // NW
CHUNK = 128
N_CHUNKS = PER_W // CHUNK
RG = CHUNK // LANES
NBUF = 4
CHUNK_ELEMS = CHUNK * EMBED_DIM


def _make_kernel():
    mesh = plsc.VectorSubcoreMesh(
        core_axis_name="c", subcore_axis_name="s", num_cores=NC, num_subcores=NS
    )

    @functools.partial(
        pl.kernel,
        out_type=jax.ShapeDtypeStruct((N_FLAT * EMBED_DIM,), jnp.float32),
        mesh=mesh,
        scratch_types=[
            pltpu.VMEM((NUM_ACTIONS * EMBED_DIM,), jnp.float32),
            pltpu.VMEM((PER_W,), jnp.int32),
            pltpu.VMEM((NBUF, CHUNK_ELEMS), jnp.float32),
            pltpu.SemaphoreType.DMA((NBUF,)),
        ],
        compiler_params=pltpu.CompilerParams(
            use_tc_tiling_on_sc=False, needs_layout_passes=False
        ),
    )
    def gather_kernel(idx_hbm, table_hbm, out_hbm, table_v, idx_v, rows_v, osem):
        wid = lax.axis_index("s") * NC + lax.axis_index("c")
        base = wid * PER_W
        pltpu.sync_copy(table_hbm, table_v)
        pltpu.sync_copy(idx_hbm.at[wid], idx_v)

        lane_row = lax.iota(jnp.int32, LANES) * EMBED_DIM
        woffs = [lane_row + rg * (LANES * EMBED_DIM) for rg in range(RG)]

        def wait_write(j, b):
            pltpu.make_async_copy(
                rows_v.at[b],
                out_hbm.at[pl.ds((base + j * CHUNK) * EMBED_DIM, CHUNK_ELEMS)],
                osem.at[b],
            ).wait()

        def body(s, carry):
            for b in range(NBUF):
                j = s * NBUF + b

                @pl.when(j >= NBUF)
                def _(j=j, b=b):
                    wait_write(j - NBUF, b)

                addrs = [
                    idx_v[pl.ds(j * CHUNK + rg * LANES, LANES)] * EMBED_DIM
                    for rg in range(RG)
                ]
                buf = rows_v.at[b]

                def col(c, carry2):
                    for rg in range(RG):
                        v = plsc.load_gather(table_v, [addrs[rg] + c])
                        plsc.store_scatter(buf, [woffs[rg] + c], v)
                    return carry2

                lax.fori_loop(0, EMBED_DIM, col, 0)
                pltpu.async_copy(
                    buf,
                    out_hbm.at[pl.ds((base + j * CHUNK) * EMBED_DIM, CHUNK_ELEMS)],
                    osem.at[b],
                )
            return carry

        lax.fori_loop(0, N_CHUNKS // NBUF, body, 0)
        for b in range(NBUF):
            wait_write(N_CHUNKS - NBUF + b, b)

    return gather_kernel


_gather = _make_kernel()


@jax.jit
def kernel(action_indices, embedding_table):
    idx = action_indices.astype(jnp.int32).reshape(NW, PER_W)
    out = _gather(idx, embedding_table.reshape(-1))
    return out.reshape(BATCH, HIST, EMBED_DIM)


# speedup vs baseline: 1.8627x; 1.8627x over previous
import functools

import jax
import jax.numpy as jnp
from jax import lax
from jax.experimental import pallas as pl
from jax.experimental.pallas import tpu as pltpu
from jax.experimental.pallas import tpu_sc as plsc

NUM_ACTIONS = 1000
EMBED_DIM = 64
BATCH = 16384
HIST = 50

NC = 2
NS = 16
NW = NC * NS
LANES = 16

N_FLAT = BATCH * HIST
PER_W = N_FLAT // NW
CHUNK = 128
N_CHUNKS = PER_W // CHUNK
RG = CHUNK // LANES
NBUF = 4
CHUNK_ELEMS = CHUNK * EMBED_DIM


def _make_kernel():
    mesh = plsc.VectorSubcoreMesh(
        core_axis_name="c", subcore_axis_name="s", num_cores=NC, num_subcores=NS
    )

    @functools.partial(
        pl.kernel,
        out_type=jax.ShapeDtypeStruct((N_FLAT * EMBED_DIM,), jnp.float32),
        mesh=mesh,
        scratch_types=[
            pltpu.VMEM((NUM_ACTIONS * EMBED_DIM,), jnp.float32),
            pltpu.VMEM((PER_W,), jnp.int32),
            pltpu.VMEM((NBUF, CHUNK_ELEMS), jnp.float32),
            pltpu.SemaphoreType.DMA((NBUF,)),
        ],
        compiler_params=pltpu.CompilerParams(
            use_tc_tiling_on_sc=False, needs_layout_passes=False
        ),
    )
    def gather_kernel(idx_hbm, table_hbm, out_hbm, table_v, idx_v, rows_v, osem):
        wid = lax.axis_index("s") * NC + lax.axis_index("c")
        base = wid * PER_W
        pltpu.sync_copy(table_hbm, table_v)
        pltpu.sync_copy(idx_hbm.at[wid], idx_v)

        lane_row = lax.iota(jnp.int32, LANES) * EMBED_DIM
        woffs = [lane_row + rg * (LANES * EMBED_DIM) for rg in range(RG)]

        def wait_write(j, b):
            pltpu.make_async_copy(
                rows_v.at[b],
                out_hbm.at[pl.ds((base + j * CHUNK) * EMBED_DIM, CHUNK_ELEMS)],
                osem.at[b],
            ).wait()

        def body(s, carry):
            for b in range(NBUF):
                j = s * NBUF + b

                @pl.when(j >= NBUF)
                def _(j=j, b=b):
                    wait_write(j - NBUF, b)

                addrs = [
                    idx_v[pl.ds(j * CHUNK + rg * LANES, LANES)] * EMBED_DIM
                    for rg in range(RG)
                ]
                buf = rows_v.at[b]

                @plsc.parallel_loop(0, EMBED_DIM, unroll=4)
                def col(c):
                    for rg in range(RG):
                        v = plsc.load_gather(table_v, [addrs[rg] + c])
                        plsc.store_scatter(buf, [woffs[rg] + c], v)
                pltpu.async_copy(
                    buf,
                    out_hbm.at[pl.ds((base + j * CHUNK) * EMBED_DIM, CHUNK_ELEMS)],
                    osem.at[b],
                )
            return carry

        lax.fori_loop(0, N_CHUNKS // NBUF, body, 0)
        for b in range(NBUF):
            wait_write(N_CHUNKS - NBUF + b, b)

    return gather_kernel


_gather = _make_kernel()


@jax.jit
def kernel(action_indices, embedding_table):
    idx = action_indices.astype(jnp.int32).reshape(NW, PER_W)
    out = _gather(idx, embedding_table.reshape(-1))
    return out.reshape(BATCH, HIST, EMBED_DIM)


# speedup vs baseline: 3.8805x; 2.0833x over previous
import functools

import jax
import jax.numpy as jnp
from jax import lax
from jax.experimental import pallas as pl
from jax.experimental.pallas import tpu as pltpu
from jax.experimental.pallas import tpu_sc as plsc

NUM_ACTIONS = 1000
EMBED_DIM = 64
BATCH = 16384
HIST = 50

NC = 2
NS = 16
NW = NC * NS

N_FLAT = BATCH * HIST
PER_W = N_FLAT // NW
CHUNK = 128
N_CHUNKS = PER_W // CHUNK
NBUF = 4
LAG = 2
N_GROUPS = -(-(N_CHUNKS + LAG) // NBUF)


def _make_kernel():
    mesh = plsc.VectorSubcoreMesh(
        core_axis_name="c", subcore_axis_name="s", num_cores=NC, num_subcores=NS
    )

    @functools.partial(
        pl.kernel,
        out_type=jax.ShapeDtypeStruct((N_FLAT, EMBED_DIM), jnp.float32),
        mesh=mesh,
        scratch_types=[
            pltpu.VMEM_SHARED((NUM_ACTIONS, EMBED_DIM), jnp.float32),
            pltpu.VMEM((N_CHUNKS, CHUNK), jnp.int32),
            pltpu.VMEM((NBUF, CHUNK, EMBED_DIM), jnp.float32),
            pltpu.SemaphoreType.DMA((NBUF,)),
            pltpu.SemaphoreType.DMA((NBUF,)),
        ],
        compiler_params=pltpu.CompilerParams(
            use_tc_tiling_on_sc=False, needs_layout_passes=False
        ),
    )
    def gather_kernel(idx_hbm, table_hbm, out_hbm, table_v, idx_v, rows_v, gsem, osem):
        sid = lax.axis_index("s")
        wid = sid * NC + lax.axis_index("c")
        base = wid * PER_W

        @pl.when(sid == 0)
        def _():
            pltpu.sync_copy(table_hbm, table_v)

        pltpu.sync_copy(idx_hbm.at[wid], idx_v)
        plsc.subcore_barrier()

        def wait_gather(j, b):
            pltpu.make_async_copy(
                table_v.at[idx_v.at[j]], rows_v.at[b], gsem.at[b]
            ).wait()

        def wait_write(j, b):
            pltpu.make_async_copy(
                rows_v.at[b], out_hbm.at[pl.ds(base + j * CHUNK, CHUNK)], osem.at[b]
            ).wait()

        def body(g, carry):
            for b in range(NBUF):
                i = g * NBUF + b

                @pl.when(i < N_CHUNKS)
                def _(i=i, b=b):
                    @pl.when(i >= NBUF)
                    def _():
                        wait_write(i - NBUF, b)

                    pltpu.async_copy(
                        table_v.at[idx_v.at[i]], rows_v.at[b], gsem.at[b]
                    )

                jw = i - LAG
                bw = (b - LAG) % NBUF

                @pl.when((jw >= 0) & (jw < N_CHUNKS))
                def _(jw=jw, bw=bw):
                    wait_gather(jw, bw)
                    pltpu.async_copy(
                        rows_v.at[bw],
                        out_hbm.at[pl.ds(base + jw * CHUNK, CHUNK)],
                        osem.at[bw],
                    )

            return carry

        lax.fori_loop(0, N_GROUPS, body, 0)

        for b in range(NBUF):
            j = N_CHUNKS - NBUF + b
            wait_write(j, j % NBUF)

    return gather_kernel


_gather = _make_kernel()


@jax.jit
def kernel(action_indices, embedding_table):
    idx = action_indices.astype(jnp.int32).reshape(NW, N_CHUNKS, CHUNK)
    out = _gather(idx, embedding_table)
    return out.reshape(BATCH, HIST, EMBED_DIM)
